# baseline (device time: 450776 ns/iter reference)
import jax
import jax.numpy as jnp
from jax import lax
from jax.experimental import pallas as pl
from jax.experimental.pallas import tpu as pltpu

N_CHUNKS = 8


def kernel(x):
    m, n = x.shape
    xb = x.astype(jnp.bfloat16)
    rows = m // N_CHUNKS

    def body(x_ref, out_ref, send_sems, recv_sems, own_sem):
        my_x = lax.axis_index("x")
        my_y = lax.axis_index("y")
        my_z = lax.axis_index("z")
        partner = (1 - my_x, my_y, my_z)

        barrier_sem = pltpu.get_barrier_semaphore()
        pl.semaphore_signal(
            barrier_sem, inc=1,
            device_id=partner, device_id_type=pl.DeviceIdType.MESH,
        )
        pl.semaphore_wait(barrier_sem, 1)

        own = pltpu.make_async_copy(
            x_ref, out_ref.at[pl.ds(my_x * m, m), :], own_sem
        )
        own.start()

        rdmas = []
        for i in range(N_CHUNKS):
            sl = pl.ds(i * rows, rows)
            dst_sl = pl.ds(my_x * m + i * rows, rows)
            rdma = pltpu.make_async_remote_copy(
                src_ref=x_ref.at[sl, :],
                dst_ref=out_ref.at[dst_sl, :],
                send_sem=send_sems.at[i],
                recv_sem=recv_sems.at[i],
                device_id=partner,
                device_id_type=pl.DeviceIdType.MESH,
            )
            rdma.start()
            rdmas.append(rdma)

        for i in range(N_CHUNKS):
            rdmas[i].wait_send()
            rdmas[i].wait_recv()
        own.wait()

    out_shape = jax.ShapeDtypeStruct((2 * m, n), jnp.bfloat16)
    return pl.pallas_call(
        body,
        out_shape=out_shape,
        in_specs=[pl.BlockSpec(memory_space=pltpu.MemorySpace.VMEM)],
        out_specs=pl.BlockSpec(memory_space=pl.ANY),
        scratch_shapes=[
            pltpu.SemaphoreType.DMA((N_CHUNKS,)),
            pltpu.SemaphoreType.DMA((N_CHUNKS,)),
            pltpu.SemaphoreType.DMA,
        ],
        compiler_params=pltpu.CompilerParams(collective_id=0),
    )(xb)


# device time: 264539 ns/iter; 1.7040x vs baseline; 1.7040x over previous
import jax
import jax.numpy as jnp
from jax import lax
from jax.experimental import pallas as pl
from jax.experimental.pallas import tpu as pltpu

NP = 8
RP = 1472
N_CW = 4
N_CCW = 3


def _ring_pos(y, z):
    return jnp.where(y == 0, z, 7 - z)


def _ring_coords(p):
    y = jnp.where(p < 4, 0, 1)
    z = jnp.where(p < 4, p, 7 - p)
    return y, z


def kernel(x):
    m, n = x.shape
    xb = x.astype(jnp.bfloat16)
    rem = m - NP * RP

    def body(
        x_ref,
        out_ref,
        ring_buf,
        inj_send_sem,
        inj_recv_sem,
        rem_send_sem,
        rem_recv_sem,
        cw_send_sems,
        cw_recv_sems,
        ccw_send_sems,
        ccw_recv_sems,
        own_sem,
        drain_sems,
        dummy_sem,
    ):
        my_x = lax.axis_index("x")
        my_y = lax.axis_index("y")
        my_z = lax.axis_index("z")
        other = 1 - my_x
        partner = (other, my_y, my_z)
        my_r = _ring_pos(my_y, my_z)
        cw_y, cw_z = _ring_coords((my_r + 1) % NP)
        ccw_y, ccw_z = _ring_coords((my_r - 1) % NP)
        cw_dev = (my_x, cw_y, cw_z)
        ccw_dev = (my_x, ccw_y, ccw_z)

        barrier_sem = pltpu.get_barrier_semaphore()
        for dev in (partner, cw_dev, ccw_dev):
            pl.semaphore_signal(
                barrier_sem, inc=1,
                device_id=dev, device_id_type=pl.DeviceIdType.MESH,
            )
        pl.semaphore_wait(barrier_sem, 3)

        own = pltpu.make_async_copy(
            x_ref, out_ref.at[pl.ds(my_x * m, m), :], own_sem
        )
        own.start()

        inj = pltpu.make_async_remote_copy(
            src_ref=x_ref.at[pl.ds(my_r * RP, RP), :],
            dst_ref=ring_buf.at[pl.ds(my_r * RP, RP), :],
            send_sem=inj_send_sem,
            recv_sem=inj_recv_sem,
            device_id=partner,
            device_id_type=pl.DeviceIdType.MESH,
        )
        inj.start()

        rem_rdma = pltpu.make_async_remote_copy(
            src_ref=x_ref.at[pl.ds(NP * RP, rem), :],
            dst_ref=out_ref.at[pl.ds(my_x * m + NP * RP, rem), :],
            send_sem=rem_send_sem,
            recv_sem=rem_recv_sem,
            device_id=partner,
            device_id_type=pl.DeviceIdType.MESH,
        )
        rem_rdma.start()

        def cw_send(s):
            q = (my_r - s) % NP
            r = pltpu.make_async_remote_copy(
                src_ref=ring_buf.at[pl.ds(q * RP, RP), :],
                dst_ref=ring_buf.at[pl.ds(q * RP, RP), :],
                send_sem=cw_send_sems.at[s],
                recv_sem=cw_recv_sems.at[s],
                device_id=cw_dev,
                device_id_type=pl.DeviceIdType.MESH,
            )
            r.start()
            return r

        def ccw_send(s):
            q = (my_r + s) % NP
            r = pltpu.make_async_remote_copy(
                src_ref=ring_buf.at[pl.ds(q * RP, RP), :],
                dst_ref=ring_buf.at[pl.ds(q * RP, RP), :],
                send_sem=ccw_send_sems.at[s],
                recv_sem=ccw_recv_sems.at[s],
                device_id=ccw_dev,
                device_id_type=pl.DeviceIdType.MESH,
            )
            r.start()
            return r

        def recv_wait(sem_arr, s, q):
            pltpu.make_async_remote_copy(
                src_ref=ring_buf.at[pl.ds(q * RP, RP), :],
                dst_ref=ring_buf.at[pl.ds(q * RP, RP), :],
                send_sem=dummy_sem,
                recv_sem=sem_arr.at[s],
                device_id=cw_dev,
                device_id_type=pl.DeviceIdType.MESH,
            ).wait_recv()

        drains = []

        def drain(q):
            d = pltpu.make_async_copy(
                ring_buf.at[pl.ds(q * RP, RP), :],
                out_ref.at[pl.ds(other * m + q * RP, RP), :],
                drain_sems.at[len(drains)],
            )
            d.start()
            drains.append(d)

        inj.wait_recv()
        drain(my_r)
        sends = [cw_send(0), ccw_send(0)]

        for s in range(1, N_CW):
            recv_wait(cw_recv_sems, s - 1, (my_r - s) % NP)
            sends.append(cw_send(s))
            drain((my_r - s) % NP)
            if s < N_CCW:
                recv_wait(ccw_recv_sems, s - 1, (my_r + s) % NP)
                sends.append(ccw_send(s))
                drain((my_r + s) % NP)
        recv_wait(cw_recv_sems, N_CW - 1, (my_r - N_CW) % NP)
        drain((my_r - N_CW) % NP)
        recv_wait(ccw_recv_sems, N_CCW - 1, (my_r + N_CCW) % NP)
        drain((my_r + N_CCW) % NP)

        rem_rdma.wait_recv()

        inj.wait_send()
        rem_rdma.wait_send()
        for snd in sends:
            snd.wait_send()
        own.wait()
        for d in drains:
            d.wait()

    out_shape = jax.ShapeDtypeStruct((2 * m, n), jnp.bfloat16)
    return pl.pallas_call(
        body,
        out_shape=out_shape,
        in_specs=[pl.BlockSpec(memory_space=pltpu.MemorySpace.VMEM)],
        out_specs=pl.BlockSpec(memory_space=pl.ANY),
        scratch_shapes=[
            pltpu.VMEM((NP * RP, n), jnp.bfloat16),
            pltpu.SemaphoreType.DMA,
            pltpu.SemaphoreType.DMA,
            pltpu.SemaphoreType.DMA,
            pltpu.SemaphoreType.DMA,
            pltpu.SemaphoreType.DMA((N_CW,)),
            pltpu.SemaphoreType.DMA((N_CW,)),
            pltpu.SemaphoreType.DMA((N_CCW,)),
            pltpu.SemaphoreType.DMA((N_CCW,)),
            pltpu.SemaphoreType.DMA,
            pltpu.SemaphoreType.DMA((NP,)),
            pltpu.SemaphoreType.DMA,
        ],
        compiler_params=pltpu.CompilerParams(
            collective_id=0, vmem_limit_bytes=60 * 1024 * 1024
        ),
    )(xb)


# device time: 237989 ns/iter; 1.8941x vs baseline; 1.1116x over previous
import jax
import jax.numpy as jnp
from jax import lax
from jax.experimental import pallas as pl
from jax.experimental.pallas import tpu as pltpu

NP = 8
NSUB = 2
RPP = 720
N_CW = 4
N_CCW = 3


def _ring_pos(y, z):
    return jnp.where(y == 0, z, 7 - z)


def _ring_coords(p):
    y = jnp.where(p < 4, 0, 1)
    z = jnp.where(p < 4, p, 7 - p)
    return y, z


def kernel(x):
    m, n = x.shape
    xb = x.astype(jnp.bfloat16)
    rp = NSUB * RPP
    ring_rows = NP * rp
    rem = m - ring_rows

    def body(
        x_ref,
        out_ref,
        ring_buf,
        inj_send_sems,
        inj_recv_sems,
        rem_send_sem,
        rem_recv_sem,
        cw_send_sems,
        cw_recv_sems,
        ccw_send_sems,
        ccw_recv_sems,
        own_sem,
        drain_sems,
        dummy_sem,
    ):
        my_x = lax.axis_index("x")
        my_y = lax.axis_index("y")
        my_z = lax.axis_index("z")
        other = 1 - my_x
        partner = (other, my_y, my_z)
        my_r = _ring_pos(my_y, my_z)
        cw_y, cw_z = _ring_coords((my_r + 1) % NP)
        ccw_y, ccw_z = _ring_coords((my_r - 1) % NP)
        cw_dev = (my_x, cw_y, cw_z)
        ccw_dev = (my_x, ccw_y, ccw_z)

        def sub_slice(piece, j):
            return pl.ds((piece * NSUB + j) * RPP, RPP)

        barrier_sem = pltpu.get_barrier_semaphore()
        for dev in (partner, cw_dev, ccw_dev):
            pl.semaphore_signal(
                barrier_sem, inc=1,
                device_id=dev, device_id_type=pl.DeviceIdType.MESH,
            )
        pl.semaphore_wait(barrier_sem, 3)

        own = pltpu.make_async_copy(
            x_ref, out_ref.at[pl.ds(my_x * m, m), :], own_sem
        )
        own.start()

        injs = []
        for j in range(NSUB):
            r = pltpu.make_async_remote_copy(
                src_ref=x_ref.at[sub_slice(my_r, j), :],
                dst_ref=ring_buf.at[sub_slice(my_r, j), :],
                send_sem=inj_send_sems.at[j],
                recv_sem=inj_recv_sems.at[j],
                device_id=partner,
                device_id_type=pl.DeviceIdType.MESH,
            )
            r.start()
            injs.append(r)

        rem_rdma = pltpu.make_async_remote_copy(
            src_ref=x_ref.at[pl.ds(ring_rows, rem), :],
            dst_ref=out_ref.at[pl.ds(my_x * m + ring_rows, rem), :],
            send_sem=rem_send_sem,
            recv_sem=rem_recv_sem,
            device_id=partner,
            device_id_type=pl.DeviceIdType.MESH,
        )
        rem_rdma.start()

        def stream_send(sem_s, sem_r, dev, piece, s, j):
            r = pltpu.make_async_remote_copy(
                src_ref=ring_buf.at[sub_slice(piece, j), :],
                dst_ref=ring_buf.at[sub_slice(piece, j), :],
                send_sem=sem_s.at[s * NSUB + j],
                recv_sem=sem_r.at[s * NSUB + j],
                device_id=dev,
                device_id_type=pl.DeviceIdType.MESH,
            )
            r.start()
            return r

        def recv_wait(sem_arr, s, j, piece):
            pltpu.make_async_remote_copy(
                src_ref=ring_buf.at[sub_slice(piece, j), :],
                dst_ref=ring_buf.at[sub_slice(piece, j), :],
                send_sem=dummy_sem,
                recv_sem=sem_arr.at[s * NSUB + j],
                device_id=cw_dev,
                device_id_type=pl.DeviceIdType.MESH,
            ).wait_recv()

        drains = []

        def drain(piece, j):
            d = pltpu.make_async_copy(
                ring_buf.at[sub_slice(piece, j), :],
                out_ref.at[pl.ds(other * m + (piece * NSUB + j) * RPP, RPP), :],
                drain_sems.at[len(drains)],
            )
            d.start()
            drains.append(d)

        sends = []
        for j in range(NSUB):
            injs[j].wait_recv()
            drain(my_r, j)
            sends.append(stream_send(
                cw_send_sems, cw_recv_sems, cw_dev, my_r, 0, j))
            sends.append(stream_send(
                ccw_send_sems, ccw_recv_sems, ccw_dev, my_r, 0, j))
        for s in range(1, N_CW):
            for j in range(NSUB):
                q = (my_r - s) % NP
                recv_wait(cw_recv_sems, s - 1, j, q)
                sends.append(stream_send(
                    cw_send_sems, cw_recv_sems, cw_dev, q, s, j))
                drain(q, j)
                if s < N_CCW:
                    qq = (my_r + s) % NP
                    recv_wait(ccw_recv_sems, s - 1, j, qq)
                    sends.append(stream_send(
                        ccw_send_sems, ccw_recv_sems, ccw_dev, qq, s, j))
                    drain(qq, j)
        for j in range(NSUB):
            recv_wait(cw_recv_sems, N_CW - 1, j, (my_r - N_CW) % NP)
            drain((my_r - N_CW) % NP, j)
            recv_wait(ccw_recv_sems, N_CCW - 1, j, (my_r + N_CCW) % NP)
            drain((my_r + N_CCW) % NP, j)

        rem_rdma.wait_recv()

        for r in injs:
            r.wait_send()
        rem_rdma.wait_send()
        for snd in sends:
            snd.wait_send()
        own.wait()
        for d in drains:
            d.wait()

    out_shape = jax.ShapeDtypeStruct((2 * m, n), jnp.bfloat16)
    return pl.pallas_call(
        body,
        out_shape=out_shape,
        in_specs=[pl.BlockSpec(memory_space=pltpu.MemorySpace.VMEM)],
        out_specs=pl.BlockSpec(memory_space=pl.ANY),
        scratch_shapes=[
            pltpu.VMEM((NP * NSUB * RPP, n), jnp.bfloat16),
            pltpu.SemaphoreType.DMA((NSUB,)),
            pltpu.SemaphoreType.DMA((NSUB,)),
            pltpu.SemaphoreType.DMA,
            pltpu.SemaphoreType.DMA,
            pltpu.SemaphoreType.DMA((N_CW * NSUB,)),
            pltpu.SemaphoreType.DMA((N_CW * NSUB,)),
            pltpu.SemaphoreType.DMA((N_CCW * NSUB,)),
            pltpu.SemaphoreType.DMA((N_CCW * NSUB,)),
            pltpu.SemaphoreType.DMA,
            pltpu.SemaphoreType.DMA((2 * NP,)),
            pltpu.SemaphoreType.DMA,
        ],
        compiler_params=pltpu.CompilerParams(
            collective_id=0, vmem_limit_bytes=60 * 1024 * 1024
        ),
    )(xb)


# device time: 208280 ns/iter; 2.1643x vs baseline; 1.1426x over previous
import jax
import jax.numpy as jnp
from jax import lax
from jax.experimental import pallas as pl
from jax.experimental.pallas import tpu as pltpu

NP = 8
NSUB = 2
RPP = 704
RP = NSUB * RPP
N_CW = 4
N_CCW = 3

RING_ROWS = NP * RP
N_ROT = 4
N_RDRAIN = 8


def _ring_pos(y, z):
    return jnp.where(y == 0, z, 7 - z)


def _ring_coords(p):
    y = jnp.where(p < 4, 0, 1)
    z = jnp.where(p < 4, p, 7 - p)
    return y, z


def kernel(x):
    m, n = x.shape
    rem_rows = m - RING_ROWS
    n_full = m // RPP
    tail = m - n_full * RPP
    chunk_lens = [RPP] * n_full + ([tail] if tail else [])
    n_chunks = len(chunk_lens)

    def body(
        x_ref,
        out_ref,
        ring_buf,
        inj_stage,
        rem_stage,
        rot_buf,
        ld_buf,
        ld_sems,
        inj_send_sems,
        inj_recv_sems,
        rem_send_sem,
        rem_recv_sem,
        cw_send_sems,
        cw_recv_sems,
        ccw_send_sems,
        ccw_recv_sems,
        ring_drain_sems,
        own_drain_sems,
        dummy_sem,
    ):
        my_x = lax.axis_index("x")
        my_y = lax.axis_index("y")
        my_z = lax.axis_index("z")
        other = 1 - my_x
        partner = (other, my_y, my_z)
        my_r = _ring_pos(my_y, my_z)
        cw_y, cw_z = _ring_coords((my_r + 1) % NP)
        ccw_y, ccw_z = _ring_coords((my_r - 1) % NP)
        cw_dev = (my_x, cw_y, cw_z)
        ccw_dev = (my_x, ccw_y, ccw_z)

        def sub_slice(piece, j):
            return pl.ds((piece * NSUB + j) * RPP, RPP)

        inj_lds = []
        for j in range(NSUB):
            ld = pltpu.make_async_copy(
                x_ref.at[pl.ds(my_r * RP + j * RPP, RPP), :],
                ld_buf.at[j],
                ld_sems.at[j],
            )
            ld.start()
            inj_lds.append(ld)

        barrier_sem = pltpu.get_barrier_semaphore()
        for dev in (partner, cw_dev, ccw_dev):
            pl.semaphore_signal(
                barrier_sem, inc=1,
                device_id=dev, device_id_type=pl.DeviceIdType.MESH,
            )
        pl.semaphore_wait(barrier_sem, 3)

        injs = []
        for j in range(NSUB):
            inj_lds[j].wait()
            inj_stage[pl.ds(j * RPP, RPP), :] = ld_buf[j].astype(jnp.bfloat16)
            r = pltpu.make_async_remote_copy(
                src_ref=inj_stage.at[pl.ds(j * RPP, RPP), :],
                dst_ref=ring_buf.at[sub_slice(my_r, j), :],
                send_sem=inj_send_sems.at[j],
                recv_sem=inj_recv_sems.at[j],
                device_id=partner,
                device_id_type=pl.DeviceIdType.MESH,
            )
            r.start()
            injs.append(r)

        chunk_lds = [None] * n_chunks
        own_drains = []

        def start_ld(k):
            ln = chunk_lens[k]
            ld = pltpu.make_async_copy(
                x_ref.at[pl.ds(k * RPP, ln), :],
                ld_buf.at[k % 2, pl.ds(0, ln), :],
                ld_sems.at[k % 2],
            )
            ld.start()
            chunk_lds[k] = ld

        start_ld(0)
        start_ld(1)
        for k in range(n_chunks):
            ln = chunk_lens[k]
            chunk_lds[k].wait()
            if k >= N_ROT:
                own_drains[k - N_ROT].wait()
            if k < 2 * NP:
                rot_buf[k % N_ROT, :, :] = ld_buf[k % 2].astype(jnp.bfloat16)
                src = rot_buf.at[k % N_ROT]
            else:
                off = (k - 2 * NP) * RPP
                rem_stage[pl.ds(off, ln), :] = (
                    ld_buf[k % 2, pl.ds(0, ln), :].astype(jnp.bfloat16)
                )
                src = rem_stage.at[pl.ds(off, ln), :]
            if k + 2 < n_chunks:
                start_ld(k + 2)
            d = pltpu.make_async_copy(
                src,
                out_ref.at[pl.ds(my_x * m + k * RPP, ln), :],
                own_drain_sems.at[k % N_ROT],
            )
            d.start()
            own_drains.append(d)

        rem_rdma = pltpu.make_async_remote_copy(
            src_ref=rem_stage,
            dst_ref=out_ref.at[pl.ds(my_x * m + RING_ROWS, rem_rows), :],
            send_sem=rem_send_sem,
            recv_sem=rem_recv_sem,
            device_id=partner,
            device_id_type=pl.DeviceIdType.MESH,
        )
        rem_rdma.start()

        def stream_send(sem_s, sem_r, dev, piece, s, j):
            r = pltpu.make_async_remote_copy(
                src_ref=ring_buf.at[sub_slice(piece, j), :],
                dst_ref=ring_buf.at[sub_slice(piece, j), :],
                send_sem=sem_s.at[s * NSUB + j],
                recv_sem=sem_r.at[s * NSUB + j],
                device_id=dev,
                device_id_type=pl.DeviceIdType.MESH,
            )
            r.start()
            return r

        def recv_wait(sem_arr, s, j, piece):
            pltpu.make_async_remote_copy(
                src_ref=ring_buf.at[sub_slice(piece, j), :],
                dst_ref=ring_buf.at[sub_slice(piece, j), :],
                send_sem=dummy_sem,
                recv_sem=sem_arr.at[s * NSUB + j],
                device_id=cw_dev,
                device_id_type=pl.DeviceIdType.MESH,
            ).wait_recv()

        ring_drains = []

        def ring_drain(piece, j):
            i = len(ring_drains)
            if i >= N_RDRAIN:
                ring_drains[i - N_RDRAIN].wait()
            d = pltpu.make_async_copy(
                ring_buf.at[sub_slice(piece, j), :],
                out_ref.at[pl.ds(other * m + (piece * NSUB + j) * RPP, RPP), :],
                ring_drain_sems.at[i % N_RDRAIN],
            )
            d.start()
            ring_drains.append(d)

        sends = []
        for j in range(NSUB):
            injs[j].wait_recv()
            ring_drain(my_r, j)
            sends.append(stream_send(
                cw_send_sems, cw_recv_sems, cw_dev, my_r, 0, j))
            sends.append(stream_send(
                ccw_send_sems, ccw_recv_sems, ccw_dev, my_r, 0, j))
        for s in range(1, N_CW):
            for j in range(NSUB):
                q = (my_r - s) % NP
                recv_wait(cw_recv_sems, s - 1, j, q)
                sends.append(stream_send(
                    cw_send_sems, cw_recv_sems, cw_dev, q, s, j))
                ring_drain(q, j)
                if s < N_CCW:
                    qq = (my_r + s) % NP
                    recv_wait(ccw_recv_sems, s - 1, j, qq)
                    sends.append(stream_send(
                        ccw_send_sems, ccw_recv_sems, ccw_dev, qq, s, j))
                    ring_drain(qq, j)
        for j in range(NSUB):
            recv_wait(cw_recv_sems, N_CW - 1, j, (my_r - N_CW) % NP)
            ring_drain((my_r - N_CW) % NP, j)
            recv_wait(ccw_recv_sems, N_CCW - 1, j, (my_r + N_CCW) % NP)
            ring_drain((my_r + N_CCW) % NP, j)

        rem_rdma.wait_recv()

        for r in injs:
            r.wait_send()
        rem_rdma.wait_send()
        for snd in sends:
            snd.wait_send()
        for d in own_drains[-N_ROT:]:
            d.wait()
        for d in ring_drains[-N_RDRAIN:]:
            d.wait()

    out_shape = jax.ShapeDtypeStruct((2 * m, n), jnp.bfloat16)
    return pl.pallas_call(
        body,
        out_shape=out_shape,
        in_specs=[pl.BlockSpec(memory_space=pl.ANY)],
        out_specs=pl.BlockSpec(memory_space=pl.ANY),
        scratch_shapes=[
            pltpu.VMEM((RING_ROWS, n), jnp.bfloat16),
            pltpu.VMEM((RP, n), jnp.bfloat16),
            pltpu.VMEM((m - RING_ROWS, n), jnp.bfloat16),
            pltpu.VMEM((N_ROT, RPP, n), jnp.bfloat16),
            pltpu.VMEM((2, RPP, n), jnp.float32),
            pltpu.SemaphoreType.DMA((2,)),
            pltpu.SemaphoreType.DMA((NSUB,)),
            pltpu.SemaphoreType.DMA((NSUB,)),
            pltpu.SemaphoreType.DMA,
            pltpu.SemaphoreType.DMA,
            pltpu.SemaphoreType.DMA((N_CW * NSUB,)),
            pltpu.SemaphoreType.DMA((N_CW * NSUB,)),
            pltpu.SemaphoreType.DMA((N_CCW * NSUB,)),
            pltpu.SemaphoreType.DMA((N_CCW * NSUB,)),
            pltpu.SemaphoreType.DMA((N_RDRAIN,)),
            pltpu.SemaphoreType.DMA((N_ROT,)),
            pltpu.SemaphoreType.DMA,
        ],
        compiler_params=pltpu.CompilerParams(
            collective_id=0, vmem_limit_bytes=56 * 1024 * 1024
        ),
    )(x)
